# baseline (speedup 1.0000x reference)
import functools

import jax
import jax.numpy as jnp
from jax import lax
from jax.experimental import pallas as pl
from jax.experimental.pallas import tpu as pltpu
from jax.experimental.pallas import tpu_sc as plsc

NUM_EXPERTS = 8
TOP_K = 2
D_MODEL = 1024
D_FF = 2048
N_TOKENS = 2048

R = 256
NT = 24
S = NT * R
T_BLK = 256
ASG = TOP_K * N_TOKENS

_N_TILES = 16
_NW = 32
_SL_W = S // _NW
_ZCH = S // _N_TILES
_GCH = 64


def _dispatch_body(x_ref, wr_ref, dest_ref, gate_ref, te_ref, tv_ref):
    T, E = N_TOKENS, NUM_EXPERTS
    logits = jnp.dot(x_ref[...], wr_ref[...], preferred_element_type=jnp.float32)
    idx = lax.broadcasted_iota(jnp.int32, (T, E), 1)
    m1 = jnp.max(logits, axis=1, keepdims=True)
    i1 = jnp.min(jnp.where(logits == m1, idx, E), axis=1, keepdims=True)
    l2 = jnp.where(idx == i1, -jnp.inf, logits)
    m2 = jnp.max(l2, axis=1, keepdims=True)
    i2 = jnp.min(jnp.where(l2 == m2, idx, E), axis=1, keepdims=True)
    e2 = jnp.exp(m2 - m1)
    g1 = 1.0 / (1.0 + e2)
    g2 = 1.0 - g1

    oh1 = (idx == i1).astype(jnp.float32)
    oh2 = (idx == i2).astype(jnp.float32)
    M = oh1 + oh2
    cs = M
    sh = 1
    while sh < T:
        cs = cs + jnp.concatenate(
            [jnp.zeros((sh, E), jnp.float32), cs[:-sh, :]], axis=0)
        sh *= 2
    cex = cs - M
    counts = jnp.sum(M, axis=0, keepdims=True)
    tiles = jnp.ceil(counts * (1.0 / R))
    ct = tiles
    sh = 1
    while sh < E:
        ct = ct + jnp.concatenate(
            [jnp.zeros((1, sh), jnp.float32), ct[:, :-sh]], axis=1)
        sh *= 2
    base_tile = ct - tiles
    total_tiles = jnp.sum(tiles)
    base = base_tile * R
    r0 = jnp.sum(oh1 * cex, axis=1, keepdims=True)
    r1 = jnp.sum(oh2 * cex, axis=1, keepdims=True)
    b0 = jnp.sum(oh1 * base, axis=1, keepdims=True)
    b1v = jnp.sum(oh2 * base, axis=1, keepdims=True)
    dest_ref[...] = jnp.concatenate([b0 + r0, b1v + r1], axis=1).astype(jnp.int32)
    gate_ref[...] = jnp.concatenate([g1, g2], axis=1)
    jrow = lax.broadcasted_iota(jnp.int32, (NT, E), 0).astype(jnp.float32)
    btb = jnp.broadcast_to(base_tile, (NT, E))
    te = jnp.sum((btb <= jrow).astype(jnp.int32), axis=1, keepdims=True) - 1
    te_ref[...] = jnp.clip(te, 0, E - 1)
    jcol = lax.broadcasted_iota(jnp.int32, (NT, 1), 0).astype(jnp.float32)
    tv_ref[...] = (jcol < total_tiles).astype(jnp.int32)


SB = 512


def _invert_body(d0_ref, d1_ref, src_ref):
    b = pl.program_id(0)
    T = N_TOKENS
    srows = b * SB + lax.broadcasted_iota(jnp.int32, (SB, 1), 0)
    cmp = ((d0_ref[...] == srows).astype(jnp.float32)
           + (d1_ref[...] == srows).astype(jnp.float32))
    tok = lax.broadcasted_iota(jnp.int32, (T, 1), 0)
    tokhl = jnp.concatenate(
        [(tok >> 6).astype(jnp.float32), (tok & 63).astype(jnp.float32)], axis=1)
    hl = jnp.dot(cmp, tokhl, preferred_element_type=jnp.float32)
    src_ref[...] = (hl[:, 0:1] * 64.0 + hl[:, 1:2]).astype(jnp.int32)


@functools.lru_cache(maxsize=None)
def _sc_kernels():
    mesh = plsc.VectorSubcoreMesh(core_axis_name="c", subcore_axis_name="s")

    nbuf = 4
    dch = _SL_W // 8
    cch = (N_TOKENS // _NW) // 4

    @functools.partial(
        pl.kernel,
        out_type=jax.ShapeDtypeStruct((S, D_MODEL), jnp.float32),
        mesh=mesh,
        scratch_types=[
            pltpu.VMEM((nbuf, dch), jnp.int32),
            pltpu.VMEM((nbuf, dch, D_MODEL), jnp.float32),
        ] + [pltpu.SemaphoreType.DMA] * nbuf,
    )
    def sc_dispatch(x_hbm, src_hbm, xs_hbm, gidx, rows, *sems):
        cid = lax.axis_index("c")
        sid = lax.axis_index("s")
        w = sid * 2 + cid
        nch = _SL_W // dch
        handles = [None] * nbuf

        def fire(q, b):
            off = w * _SL_W + q * dch
            pltpu.sync_copy(src_hbm.at[pl.ds(off, dch)], gidx.at[b])
            return pltpu.async_copy(x_hbm.at[gidx.at[b]], rows.at[b], sems[b])

        for b in range(nbuf):
            handles[b] = fire(b, b)
        for q in range(nch):
            b = q % nbuf
            handles[b].wait()
            pltpu.sync_copy(rows.at[b], xs_hbm.at[pl.ds(w * _SL_W + q * dch, dch)])
            if q + nbuf < nch:
                handles[b] = fire(q + nbuf, b)

    @functools.partial(
        pl.kernel,
        out_type=(jax.ShapeDtypeStruct((N_TOKENS, D_MODEL), jnp.float32),
                  jax.ShapeDtypeStruct((N_TOKENS, D_MODEL), jnp.float32)),
        mesh=mesh,
        scratch_types=[
            pltpu.VMEM((nbuf, cch), jnp.int32),
            pltpu.VMEM((nbuf, cch, D_MODEL), jnp.float32),
        ] + [pltpu.SemaphoreType.DMA] * nbuf,
    )
    def sc_combine(ys_hbm, d0_hbm, d1_hbm, y0_hbm, y1_hbm, gidx, rows, *sems):
        cid = lax.axis_index("c")
        sid = lax.axis_index("s")
        w = sid * 2 + cid
        base = w * (N_TOKENS // _NW)
        nch_k = (N_TOKENS // _NW) // cch
        chunks = [(k, c) for k in range(2) for c in range(nch_k)]
        handles = [None] * nbuf

        def fire(i, b):
            k, c = chunks[i]
            dk = d0_hbm if k == 0 else d1_hbm
            pltpu.sync_copy(dk.at[pl.ds(base + c * cch, cch)], gidx.at[b])
            return pltpu.async_copy(ys_hbm.at[gidx.at[b]], rows.at[b], sems[b])

        for b in range(nbuf):
            handles[b] = fire(b, b)
        for i in range(len(chunks)):
            b = i % nbuf
            k, c = chunks[i]
            yk = y0_hbm if k == 0 else y1_hbm
            handles[b].wait()
            pltpu.sync_copy(rows.at[b], yk.at[pl.ds(base + c * cch, cch)])
            if i + nbuf < len(chunks):
                handles[b] = fire(i + nbuf, b)

    return sc_dispatch, sc_combine


def _gffn_body(te_ref, tv_ref, xs_ref, w1_ref, b1_ref, w2_ref, b2_ref, ys_ref):
    j = pl.program_id(0)

    @pl.when(tv_ref[j] == 1)
    def _():
        h = jnp.dot(xs_ref[...].astype(jnp.bfloat16),
                    w1_ref[0].astype(jnp.bfloat16),
                    preferred_element_type=jnp.float32) + b1_ref[0]
        h = jax.nn.gelu(h)
        ys_ref[...] = jnp.dot(h.astype(jnp.bfloat16),
                              w2_ref[0].astype(jnp.bfloat16),
                              preferred_element_type=jnp.float32) + b2_ref[0]


def _combine_body(y0_ref, y1_ref, g0_ref, g1_ref, o_ref):
    o_ref[...] = g0_ref[...] * y0_ref[...] + g1_ref[...] * y1_ref[...]


def kernel(hidden_states, Wr, W1, b1, W2, b2):
    dest, gates, te, tv = pl.pallas_call(
        _dispatch_body,
        out_shape=(
            jax.ShapeDtypeStruct((N_TOKENS, TOP_K), jnp.int32),
            jax.ShapeDtypeStruct((N_TOKENS, TOP_K), jnp.float32),
            jax.ShapeDtypeStruct((NT, 1), jnp.int32),
            jax.ShapeDtypeStruct((NT, 1), jnp.int32),
        ),
    )(hidden_states, Wr)

    te1 = te.reshape(NT)
    tv1 = tv.reshape(NT)

    src = pl.pallas_call(
        _invert_body,
        grid=(S // SB,),
        in_specs=[
            pl.BlockSpec((1, N_TOKENS), lambda b: (0, 0)),
            pl.BlockSpec((1, N_TOKENS), lambda b: (0, 0)),
        ],
        out_specs=pl.BlockSpec((SB, 1), lambda b: (b, 0)),
        out_shape=jax.ShapeDtypeStruct((S, 1), jnp.int32),
    )(dest[:, 0].reshape(1, N_TOKENS), dest[:, 1].reshape(1, N_TOKENS))

    sc_dispatch, sc_combine = _sc_kernels()
    xs = sc_dispatch(hidden_states, src.reshape(S))

    ys = pl.pallas_call(
        _gffn_body,
        grid_spec=pltpu.PrefetchScalarGridSpec(
            num_scalar_prefetch=2,
            grid=(NT,),
            in_specs=[
                pl.BlockSpec((R, D_MODEL), lambda j, te_, tv_: (j, 0)),
                pl.BlockSpec((1, D_MODEL, D_FF), lambda j, te_, tv_: (te_[j], 0, 0)),
                pl.BlockSpec((1, 1, D_FF), lambda j, te_, tv_: (te_[j], 0, 0)),
                pl.BlockSpec((1, D_FF, D_MODEL), lambda j, te_, tv_: (te_[j], 0, 0)),
                pl.BlockSpec((1, 1, D_MODEL), lambda j, te_, tv_: (te_[j], 0, 0)),
            ],
            out_specs=pl.BlockSpec((R, D_MODEL), lambda j, te_, tv_: (j, 0)),
        ),
        out_shape=jax.ShapeDtypeStruct((S, D_MODEL), jnp.float32),
    )(te1, tv1, xs, W1, b1[:, None, :], W2, b2[:, None, :])

    y0, y1 = sc_combine(ys, dest[:, 0], dest[:, 1])

    out = pl.pallas_call(
        _combine_body,
        grid=(N_TOKENS // T_BLK,),
        in_specs=[
            pl.BlockSpec((T_BLK, D_MODEL), lambda t: (t, 0)),
            pl.BlockSpec((T_BLK, D_MODEL), lambda t: (t, 0)),
            pl.BlockSpec((T_BLK, 1), lambda t: (t, 0)),
            pl.BlockSpec((T_BLK, 1), lambda t: (t, 0)),
        ],
        out_specs=pl.BlockSpec((T_BLK, D_MODEL), lambda t: (t, 0)),
        out_shape=jax.ShapeDtypeStruct((N_TOKENS, D_MODEL), jnp.float32),
    )(y0, y1, gates[:, 0:1], gates[:, 1:2])
    return out


# speedup vs baseline: 1.5405x; 1.5405x over previous
import functools

import jax
import jax.numpy as jnp
from jax import lax
from jax.experimental import pallas as pl
from jax.experimental.pallas import tpu as pltpu
from jax.experimental.pallas import tpu_sc as plsc

NUM_EXPERTS = 8
TOP_K = 2
D_MODEL = 1024
D_FF = 2048
N_TOKENS = 2048

R = 256
NT = 24
S = NT * R
T_BLK = 256
ASG = TOP_K * N_TOKENS

_N_TILES = 16
_NW = 32
_SL_W = S // _NW
_ZCH = S // _N_TILES
_GCH = 64


def _dispatch_body(x_ref, wr_ref, dest_ref, gate_ref, te_ref, tv_ref):
    T, E = N_TOKENS, NUM_EXPERTS
    logits = jnp.dot(x_ref[...], wr_ref[...], preferred_element_type=jnp.float32)
    idx = lax.broadcasted_iota(jnp.int32, (T, E), 1)
    m1 = jnp.max(logits, axis=1, keepdims=True)
    i1 = jnp.min(jnp.where(logits == m1, idx, E), axis=1, keepdims=True)
    l2 = jnp.where(idx == i1, -jnp.inf, logits)
    m2 = jnp.max(l2, axis=1, keepdims=True)
    i2 = jnp.min(jnp.where(l2 == m2, idx, E), axis=1, keepdims=True)
    e2 = jnp.exp(m2 - m1)
    g1 = 1.0 / (1.0 + e2)
    g2 = 1.0 - g1

    oh1 = (idx == i1).astype(jnp.float32)
    oh2 = (idx == i2).astype(jnp.float32)
    M = oh1 + oh2
    cs = M
    sh = 1
    while sh < T:
        cs = cs + jnp.concatenate(
            [jnp.zeros((sh, E), jnp.float32), cs[:-sh, :]], axis=0)
        sh *= 2
    cex = cs - M
    counts = jnp.sum(M, axis=0, keepdims=True)
    tiles = jnp.ceil(counts * (1.0 / R))
    ct = tiles
    sh = 1
    while sh < E:
        ct = ct + jnp.concatenate(
            [jnp.zeros((1, sh), jnp.float32), ct[:, :-sh]], axis=1)
        sh *= 2
    base_tile = ct - tiles
    total_tiles = jnp.sum(tiles)
    base = base_tile * R
    r0 = jnp.sum(oh1 * cex, axis=1, keepdims=True)
    r1 = jnp.sum(oh2 * cex, axis=1, keepdims=True)
    b0 = jnp.sum(oh1 * base, axis=1, keepdims=True)
    b1v = jnp.sum(oh2 * base, axis=1, keepdims=True)
    dest_ref[...] = jnp.concatenate([b0 + r0, b1v + r1], axis=1).astype(jnp.int32)
    gate_ref[...] = jnp.concatenate([g1, g2], axis=1)
    jrow = lax.broadcasted_iota(jnp.int32, (NT, E), 0).astype(jnp.float32)
    btb = jnp.broadcast_to(base_tile, (NT, E))
    te = jnp.sum((btb <= jrow).astype(jnp.int32), axis=1, keepdims=True) - 1
    te_ref[...] = jnp.clip(te, 0, E - 1)
    jcol = lax.broadcasted_iota(jnp.int32, (NT, 1), 0).astype(jnp.float32)
    tv_ref[...] = (jcol < total_tiles).astype(jnp.int32)


SB = 512


def _invert_body(d0_ref, d1_ref, src_ref):
    b = pl.program_id(0)
    T = N_TOKENS
    srows = b * SB + lax.broadcasted_iota(jnp.int32, (SB, 1), 0)
    cmp = ((d0_ref[...] == srows).astype(jnp.float32)
           + (d1_ref[...] == srows).astype(jnp.float32))
    tok = lax.broadcasted_iota(jnp.int32, (T, 1), 0)
    tokhl = jnp.concatenate(
        [(tok >> 6).astype(jnp.float32), (tok & 63).astype(jnp.float32),
         jnp.ones((T, 1), jnp.float32)], axis=1)
    hl = jnp.dot(cmp, tokhl, preferred_element_type=jnp.float32)
    pad_row = (srows & (T - 1)).astype(jnp.float32) * (1.0 - hl[:, 2:3])
    src_ref[...] = (hl[:, 0:1] * 64.0 + hl[:, 1:2] + pad_row).astype(jnp.int32)


@functools.lru_cache(maxsize=None)
def _sc_kernels():
    mesh = plsc.VectorSubcoreMesh(core_axis_name="c", subcore_axis_name="s")

    nbuf = 4
    dch = _SL_W // 8
    cch = (N_TOKENS // _NW) // 4

    @functools.partial(
        pl.kernel,
        out_type=jax.ShapeDtypeStruct((S, D_MODEL), jnp.float32),
        mesh=mesh,
        scratch_types=[
            pltpu.VMEM((nbuf, dch), jnp.int32),
            pltpu.VMEM((nbuf, dch, D_MODEL), jnp.float32),
        ] + [pltpu.SemaphoreType.DMA] * nbuf,
    )
    def sc_dispatch(x_hbm, src_hbm, xs_hbm, gidx, rows, *sems):
        cid = lax.axis_index("c")
        sid = lax.axis_index("s")
        w = sid * 2 + cid
        nch = _SL_W // dch
        handles = [None] * nbuf

        def fire(q, b):
            off = w * _SL_W + q * dch
            pltpu.sync_copy(src_hbm.at[pl.ds(off, dch)], gidx.at[b])
            return pltpu.async_copy(x_hbm.at[gidx.at[b]], rows.at[b], sems[b])

        for b in range(nbuf):
            handles[b] = fire(b, b)
        for q in range(nch):
            b = q % nbuf
            handles[b].wait()
            pltpu.sync_copy(rows.at[b], xs_hbm.at[pl.ds(w * _SL_W + q * dch, dch)])
            if q + nbuf < nch:
                handles[b] = fire(q + nbuf, b)

    @functools.partial(
        pl.kernel,
        out_type=(jax.ShapeDtypeStruct((N_TOKENS, D_MODEL), jnp.float32),
                  jax.ShapeDtypeStruct((N_TOKENS, D_MODEL), jnp.float32)),
        mesh=mesh,
        scratch_types=[
            pltpu.VMEM((nbuf, cch), jnp.int32),
            pltpu.VMEM((nbuf, cch, D_MODEL), jnp.float32),
        ] + [pltpu.SemaphoreType.DMA] * nbuf,
    )
    def sc_combine(ys_hbm, d0_hbm, d1_hbm, y0_hbm, y1_hbm, gidx, rows, *sems):
        cid = lax.axis_index("c")
        sid = lax.axis_index("s")
        w = sid * 2 + cid
        base = w * (N_TOKENS // _NW)
        nch_k = (N_TOKENS // _NW) // cch
        chunks = [(k, c) for k in range(2) for c in range(nch_k)]
        handles = [None] * nbuf

        def fire(i, b):
            k, c = chunks[i]
            dk = d0_hbm if k == 0 else d1_hbm
            pltpu.sync_copy(dk.at[pl.ds(base + c * cch, cch)], gidx.at[b])
            return pltpu.async_copy(ys_hbm.at[gidx.at[b]], rows.at[b], sems[b])

        for b in range(nbuf):
            handles[b] = fire(b, b)
        for i in range(len(chunks)):
            b = i % nbuf
            k, c = chunks[i]
            yk = y0_hbm if k == 0 else y1_hbm
            handles[b].wait()
            pltpu.sync_copy(rows.at[b], yk.at[pl.ds(base + c * cch, cch)])
            if i + nbuf < len(chunks):
                handles[b] = fire(i + nbuf, b)

    return sc_dispatch, sc_combine


def _gffn_body(te_ref, tv_ref, xs_ref, w1_ref, b1_ref, w2_ref, b2_ref, ys_ref):
    j = pl.program_id(0)

    @pl.when(tv_ref[j] == 1)
    def _():
        h = jnp.dot(xs_ref[...].astype(jnp.bfloat16),
                    w1_ref[0].astype(jnp.bfloat16),
                    preferred_element_type=jnp.float32) + b1_ref[0]
        h = jax.nn.gelu(h)
        ys_ref[...] = jnp.dot(h.astype(jnp.bfloat16),
                              w2_ref[0].astype(jnp.bfloat16),
                              preferred_element_type=jnp.float32) + b2_ref[0]


def _combine_body(y0_ref, y1_ref, g0_ref, g1_ref, o_ref):
    o_ref[...] = g0_ref[...] * y0_ref[...] + g1_ref[...] * y1_ref[...]


def kernel(hidden_states, Wr, W1, b1, W2, b2):
    dest, gates, te, tv = pl.pallas_call(
        _dispatch_body,
        out_shape=(
            jax.ShapeDtypeStruct((N_TOKENS, TOP_K), jnp.int32),
            jax.ShapeDtypeStruct((N_TOKENS, TOP_K), jnp.float32),
            jax.ShapeDtypeStruct((NT, 1), jnp.int32),
            jax.ShapeDtypeStruct((NT, 1), jnp.int32),
        ),
    )(hidden_states, Wr)

    te1 = te.reshape(NT)
    tv1 = tv.reshape(NT)

    src = pl.pallas_call(
        _invert_body,
        grid=(S // SB,),
        in_specs=[
            pl.BlockSpec((1, N_TOKENS), lambda b: (0, 0)),
            pl.BlockSpec((1, N_TOKENS), lambda b: (0, 0)),
        ],
        out_specs=pl.BlockSpec((SB, 1), lambda b: (b, 0)),
        out_shape=jax.ShapeDtypeStruct((S, 1), jnp.int32),
    )(dest[:, 0].reshape(1, N_TOKENS), dest[:, 1].reshape(1, N_TOKENS))

    sc_dispatch, sc_combine = _sc_kernels()
    xs = sc_dispatch(hidden_states, src.reshape(S))

    ys = pl.pallas_call(
        _gffn_body,
        grid_spec=pltpu.PrefetchScalarGridSpec(
            num_scalar_prefetch=2,
            grid=(NT,),
            in_specs=[
                pl.BlockSpec((R, D_MODEL), lambda j, te_, tv_: (j, 0)),
                pl.BlockSpec((1, D_MODEL, D_FF), lambda j, te_, tv_: (te_[j], 0, 0)),
                pl.BlockSpec((1, 1, D_FF), lambda j, te_, tv_: (te_[j], 0, 0)),
                pl.BlockSpec((1, D_FF, D_MODEL), lambda j, te_, tv_: (te_[j], 0, 0)),
                pl.BlockSpec((1, 1, D_MODEL), lambda j, te_, tv_: (te_[j], 0, 0)),
            ],
            out_specs=pl.BlockSpec((R, D_MODEL), lambda j, te_, tv_: (j, 0)),
        ),
        out_shape=jax.ShapeDtypeStruct((S, D_MODEL), jnp.float32),
    )(te1, tv1, xs, W1, b1[:, None, :], W2, b2[:, None, :])

    y0, y1 = sc_combine(ys, dest[:, 0], dest[:, 1])

    out = pl.pallas_call(
        _combine_body,
        grid=(N_TOKENS // T_BLK,),
        in_specs=[
            pl.BlockSpec((T_BLK, D_MODEL), lambda t: (t, 0)),
            pl.BlockSpec((T_BLK, D_MODEL), lambda t: (t, 0)),
            pl.BlockSpec((T_BLK, 1), lambda t: (t, 0)),
            pl.BlockSpec((T_BLK, 1), lambda t: (t, 0)),
        ],
        out_specs=pl.BlockSpec((T_BLK, D_MODEL), lambda t: (t, 0)),
        out_shape=jax.ShapeDtypeStruct((N_TOKENS, D_MODEL), jnp.float32),
    )(y0, y1, gates[:, 0:1], gates[:, 1:2])
    return out
